# baseline (speedup 1.0000x reference)
import functools

import jax
import jax.numpy as jnp
from jax import lax
from jax.experimental import pallas as pl
from jax.experimental.pallas import tpu as pltpu
from jax.experimental.pallas import tpu_sc as plsc

N_NODES = 10000
N_EDGES = 320000
D = 128

NC = 2
NS = 16
NW = NC * NS

K = 96
CHUNKS = 106
HC = CHUNKS // 2
NBUF = 3
E_PAD = NW * CHUNKS * K
N_PAD = 10112
JUNK = N_PAD - N_NODES
ROWS_PER_TILE = N_PAD // NS
DEG_BINS = 16384
DEG_PER_TILE = DEG_BINS // NS

LAST_ROWS = N_NODES - (NS - 1) * ROWS_PER_TILE

BG = 2000
GRID = N_NODES // BG

_sc_mesh = plsc.VectorSubcoreMesh(
    core_axis_name="c", subcore_axis_name="s", num_cores=NC, num_subcores=NS)


@functools.partial(
    pl.kernel,
    out_type=jax.ShapeDtypeStruct((NC, DEG_BINS), jnp.float32),
    mesh=_sc_mesh,
    scratch_types=[
        pltpu.VMEM((HC, K), jnp.int32),
        pltpu.VMEM((K,), jnp.float32),
        pltpu.VMEM((DEG_PER_TILE,), jnp.float32),
        pltpu.VMEM_SHARED((DEG_BINS,), jnp.float32),
    ],
)
def _deg_kernel(dst_hbm, out_hbm, dst_v, ones_v, z_v, hist_sh):
    c = lax.axis_index("c")
    s = lax.axis_index("s")
    wid = s * NC + c
    for i in range(K // 16):
        ones_v[pl.ds(i * 16, 16)] = jnp.ones((16,), jnp.float32)
    for i in range(DEG_PER_TILE // 16):
        z_v[pl.ds(i * 16, 16)] = jnp.zeros((16,), jnp.float32)
    pltpu.sync_copy(z_v, hist_sh.at[pl.ds(s * DEG_PER_TILE, DEG_PER_TILE)])
    plsc.subcore_barrier()

    for h in range(2):
        pltpu.sync_copy(dst_hbm.at[wid * 2 + h], dst_v)

        @pl.loop(0, HC)
        def _(j):
            pltpu.sync_copy(ones_v, hist_sh.at[dst_v.at[j]], add=True)

    plsc.subcore_barrier()
    pltpu.sync_copy(hist_sh.at[pl.ds(s * DEG_PER_TILE, DEG_PER_TILE)],
                    out_hbm.at[c, pl.ds(s * DEG_PER_TILE, DEG_PER_TILE)])


@functools.partial(
    pl.kernel,
    out_type=jax.ShapeDtypeStruct((NC, N_PAD, D), jnp.float32),
    mesh=_sc_mesh,
    scratch_types=[
        pltpu.VMEM((HC * K,), jnp.int32),
        pltpu.VMEM((HC, K), jnp.int32),
        pltpu.VMEM((NBUF, K, D), jnp.float32),
        pltpu.VMEM_SHARED((N_PAD, D), jnp.float32),
        pltpu.SemaphoreType.DMA,
        pltpu.SemaphoreType.DMA,
        pltpu.SemaphoreType.DMA,
    ],
)
def _agg_kernel(g_hbm, z_hbm, src_hbm, dst_hbm, out_hbm,
                src_v, dst_v, rows_v, acc_sh, gsem0, gsem1, gsem2):
    c = lax.axis_index("c")
    s = lax.axis_index("s")
    wid = s * NC + c
    base = s * ROWS_PER_TILE

    @pl.when((c == 0) & (s < NS - 1))
    def _():
        pltpu.sync_copy(g_hbm.at[pl.ds(base, ROWS_PER_TILE)],
                        acc_sh.at[pl.ds(base, ROWS_PER_TILE)])

    @pl.when((c == 0) & (s == NS - 1))
    def _():
        pltpu.sync_copy(g_hbm.at[pl.ds(base, LAST_ROWS)],
                        acc_sh.at[pl.ds(base, LAST_ROWS)])

    @pl.when(c != 0)
    def _():
        for t in range(ROWS_PER_TILE // K):
            pltpu.sync_copy(z_hbm, acc_sh.at[pl.ds(base + t * K, K)])
        rem = ROWS_PER_TILE % K
        pltpu.sync_copy(
            z_hbm.at[pl.ds(0, rem)],
            acc_sh.at[pl.ds(base + (ROWS_PER_TILE // K) * K, rem)])

    plsc.subcore_barrier()

    gsems = (gsem0, gsem1, gsem2)
    for h in range(2):
        pltpu.sync_copy(src_hbm.at[wid * 2 + h], src_v)
        pltpu.sync_copy(dst_hbm.at[wid * 2 + h], dst_v)
        for b in range(NBUF):
            pltpu.async_copy(
                g_hbm.at[src_v.at[pl.ds(b * K, K)]], rows_v.at[b], gsems[b])

        @pl.loop(0, HC - (HC % NBUF), step=NBUF)
        def _(j):
            for b in range(NBUF):
                jj = j + b
                pltpu.make_async_copy(
                    g_hbm.at[src_v.at[pl.ds(jj * K, K)]],
                    rows_v.at[b], gsems[b]).wait()
                pltpu.sync_copy(rows_v.at[b], acc_sh.at[dst_v.at[jj]], add=True)

                @pl.when(jj + NBUF < HC)
                def _():
                    pltpu.async_copy(
                        g_hbm.at[src_v.at[pl.ds((jj + NBUF) * K, K)]],
                        rows_v.at[b], gsems[b])

        for b in range(HC % NBUF):
            jj = HC - (HC % NBUF) + b
            pltpu.make_async_copy(
                g_hbm.at[src_v.at[pl.ds(jj * K, K)]],
                rows_v.at[b], gsems[b]).wait()
            pltpu.sync_copy(rows_v.at[b], acc_sh.at[dst_v.at[jj]], add=True)

    plsc.subcore_barrier()
    pltpu.sync_copy(acc_sh.at[pl.ds(base, ROWS_PER_TILE)],
                    out_hbm.at[c, pl.ds(base, ROWS_PER_TILE)])


def _mm_first_body(x_ref, c0_ref, c1_ref, w_ref, g_ref, dis_ref):
    deg = c0_ref[0] + c1_ref[0] + 1.0
    dis = lax.rsqrt(deg)
    dis_ref[...] = dis
    g_ref[...] = dis * jnp.dot(x_ref[...], w_ref[...],
                               preferred_element_type=jnp.float32)


_mm_first = pl.pallas_call(
    _mm_first_body,
    grid=(GRID,),
    in_specs=[
        pl.BlockSpec((BG, D), lambda i: (i, 0)),
        pl.BlockSpec((1, BG, 1), lambda i: (0, i, 0)),
        pl.BlockSpec((1, BG, 1), lambda i: (1, i, 0)),
        pl.BlockSpec((D, D), lambda i: (0, 0)),
    ],
    out_specs=[
        pl.BlockSpec((BG, D), lambda i: (i, 0)),
        pl.BlockSpec((BG, 1), lambda i: (i, 0)),
    ],
    out_shape=[
        jax.ShapeDtypeStruct((N_NODES, D), jnp.float32),
        jax.ShapeDtypeStruct((N_NODES, 1), jnp.float32),
    ],
)


def _mm_mid_body(z0_ref, z1_ref, dis_ref, b_ref, w_ref, g_ref, *, relu):
    dis = dis_ref[...]
    u = dis * (z0_ref[0] + z1_ref[0]) + b_ref[...]
    h = jnp.maximum(u, 0.0) if relu else u
    g_ref[...] = dis * jnp.dot(h, w_ref[...],
                               preferred_element_type=jnp.float32)


def _make_mm_mid(relu):
    return pl.pallas_call(
        functools.partial(_mm_mid_body, relu=relu),
        grid=(GRID,),
        in_specs=[
            pl.BlockSpec((1, BG, D), lambda i: (0, i, 0)),
            pl.BlockSpec((1, BG, D), lambda i: (1, i, 0)),
            pl.BlockSpec((BG, 1), lambda i: (i, 0)),
            pl.BlockSpec((1, D), lambda i: (0, 0)),
            pl.BlockSpec((D, D), lambda i: (0, 0)),
        ],
        out_specs=pl.BlockSpec((BG, D), lambda i: (i, 0)),
        out_shape=jax.ShapeDtypeStruct((N_NODES, D), jnp.float32),
    )


_mm_mid_relu = _make_mm_mid(True)
_mm_mid_lin = _make_mm_mid(False)


def _mm_last_body(z0_ref, z1_ref, dis_ref, b_ref, o_ref):
    o_ref[...] = dis_ref[...] * (z0_ref[0] + z1_ref[0]) + b_ref[...]


_mm_last = pl.pallas_call(
    _mm_last_body,
    grid=(GRID,),
    in_specs=[
        pl.BlockSpec((1, BG, D), lambda i: (0, i, 0)),
        pl.BlockSpec((1, BG, D), lambda i: (1, i, 0)),
        pl.BlockSpec((BG, 1), lambda i: (i, 0)),
        pl.BlockSpec((1, D), lambda i: (0, 0)),
    ],
    out_specs=pl.BlockSpec((BG, D), lambda i: (i, 0)),
    out_shape=jax.ShapeDtypeStruct((N_NODES, D), jnp.float32),
)


def kernel(x, edge_index, W1, b1, W2, b2):
    src = edge_index[0].astype(jnp.int32)
    dst = edge_index[1].astype(jnp.int32)
    npad = E_PAD - N_EDGES
    pidx = jnp.arange(npad, dtype=jnp.int32)
    src_full = jnp.concatenate([src, pidx % N_NODES])
    dst_full = jnp.concatenate([dst, N_NODES + pidx % JUNK])
    src3 = src_full.reshape(NW * 2, HC * K)
    dst3 = dst_full.reshape(NW * 2, HC, K)

    cnt = _deg_kernel(dst3)
    cnt3 = cnt[:, :, None]
    zeros = jnp.zeros((K, D), jnp.float32)
    b1r = b1.reshape(1, D)
    b2r = b2.reshape(1, D)

    g1, dis = _mm_first(x, cnt3, cnt3, W1)
    z1 = _agg_kernel(g1, zeros, src3, dst3)
    g2 = _mm_mid_relu(z1, z1, dis, b1r, W2)
    z2 = _agg_kernel(g2, zeros, src3, dst3)
    g3 = _mm_mid_lin(z2, z2, dis, b2r, W1)
    z3 = _agg_kernel(g3, zeros, src3, dst3)
    return _mm_last(z3, z3, dis, b1r)


# speedup vs baseline: 1.0288x; 1.0288x over previous
import functools

import jax
import jax.numpy as jnp
from jax import lax
from jax.experimental import pallas as pl
from jax.experimental.pallas import tpu as pltpu
from jax.experimental.pallas import tpu_sc as plsc

N_NODES = 10000
N_EDGES = 320000
D = 128

NC = 2
NS = 16
NW = NC * NS

K = 96
CHUNKS = 106
HC = CHUNKS // 2
NBUF = 3
E_PAD = NW * CHUNKS * K
N_PAD = 10112
JUNK = N_PAD - N_NODES
ROWS_PER_TILE = N_PAD // NS
DEG_BINS = 16384
DEG_PER_TILE = DEG_BINS // NS

LAST_ROWS = N_NODES - (NS - 1) * ROWS_PER_TILE

BG = 2000
GRID = N_NODES // BG

_sc_mesh = plsc.VectorSubcoreMesh(
    core_axis_name="c", subcore_axis_name="s", num_cores=NC, num_subcores=NS)


@functools.partial(
    pl.kernel,
    out_type=jax.ShapeDtypeStruct((NC, DEG_BINS), jnp.float32),
    mesh=_sc_mesh,
    scratch_types=[
        pltpu.VMEM((HC, K), jnp.int32),
        pltpu.VMEM((K,), jnp.float32),
        pltpu.VMEM((DEG_PER_TILE,), jnp.float32),
        pltpu.VMEM_SHARED((DEG_BINS,), jnp.float32),
    ],
)
def _deg_kernel(dst_hbm, out_hbm, dst_v, ones_v, z_v, hist_sh):
    c = lax.axis_index("c")
    s = lax.axis_index("s")
    wid = s * NC + c
    for i in range(K // 16):
        ones_v[pl.ds(i * 16, 16)] = jnp.ones((16,), jnp.float32)
    for i in range(DEG_PER_TILE // 16):
        z_v[pl.ds(i * 16, 16)] = jnp.zeros((16,), jnp.float32)
    pltpu.sync_copy(z_v, hist_sh.at[pl.ds(s * DEG_PER_TILE, DEG_PER_TILE)])
    plsc.subcore_barrier()

    for h in range(2):
        pltpu.sync_copy(dst_hbm.at[wid * 2 + h], dst_v)

        @pl.loop(0, HC)
        def _(j):
            pltpu.sync_copy(ones_v, hist_sh.at[dst_v.at[j]], add=True)

    plsc.subcore_barrier()
    pltpu.sync_copy(hist_sh.at[pl.ds(s * DEG_PER_TILE, DEG_PER_TILE)],
                    out_hbm.at[c, pl.ds(s * DEG_PER_TILE, DEG_PER_TILE)])


@functools.partial(
    pl.kernel,
    out_type=jax.ShapeDtypeStruct((NC, N_PAD, D), jnp.float32),
    mesh=_sc_mesh,
    scratch_types=[
        pltpu.VMEM((HC * K,), jnp.int32),
        pltpu.VMEM((HC, K), jnp.int32),
        pltpu.VMEM((NBUF, K, D), jnp.float32),
        pltpu.VMEM_SHARED((N_PAD, D), jnp.float32),
        pltpu.SemaphoreType.DMA,
        pltpu.SemaphoreType.DMA,
        pltpu.SemaphoreType.DMA,
    ],
)
def _agg_kernel(g_hbm, z_hbm, src_hbm, dst_hbm, out_hbm,
                src_v, dst_v, rows_v, acc_sh, gsem0, gsem1, gsem2):
    c = lax.axis_index("c")
    s = lax.axis_index("s")
    wid = s * NC + c
    base = s * ROWS_PER_TILE

    @pl.when((c == 0) & (s < NS - 1))
    def _():
        pltpu.sync_copy(g_hbm.at[pl.ds(base, ROWS_PER_TILE)],
                        acc_sh.at[pl.ds(base, ROWS_PER_TILE)])

    @pl.when((c == 0) & (s == NS - 1))
    def _():
        pltpu.sync_copy(g_hbm.at[pl.ds(base, LAST_ROWS)],
                        acc_sh.at[pl.ds(base, LAST_ROWS)])

    @pl.when((c != 0) & (s < NS - 1))
    def _():
        pltpu.sync_copy(z_hbm.at[pl.ds(base, ROWS_PER_TILE)],
                        acc_sh.at[pl.ds(base, ROWS_PER_TILE)])

    @pl.when((c != 0) & (s == NS - 1))
    def _():
        pltpu.sync_copy(z_hbm.at[pl.ds(base, LAST_ROWS)],
                        acc_sh.at[pl.ds(base, LAST_ROWS)])

    plsc.subcore_barrier()

    gsems = (gsem0, gsem1, gsem2)
    for h in range(2):
        pltpu.sync_copy(src_hbm.at[wid * 2 + h], src_v)
        pltpu.sync_copy(dst_hbm.at[wid * 2 + h], dst_v)
        for b in range(NBUF):
            pltpu.async_copy(
                g_hbm.at[src_v.at[pl.ds(b * K, K)]], rows_v.at[b], gsems[b])

        @pl.loop(0, HC - (HC % NBUF), step=NBUF)
        def _(j):
            for b in range(NBUF):
                jj = j + b
                pltpu.make_async_copy(
                    g_hbm.at[src_v.at[pl.ds(jj * K, K)]],
                    rows_v.at[b], gsems[b]).wait()
                pltpu.sync_copy(rows_v.at[b], acc_sh.at[dst_v.at[jj]], add=True)

                @pl.when(jj + NBUF < HC)
                def _():
                    pltpu.async_copy(
                        g_hbm.at[src_v.at[pl.ds((jj + NBUF) * K, K)]],
                        rows_v.at[b], gsems[b])

        for b in range(HC % NBUF):
            jj = HC - (HC % NBUF) + b
            pltpu.make_async_copy(
                g_hbm.at[src_v.at[pl.ds(jj * K, K)]],
                rows_v.at[b], gsems[b]).wait()
            pltpu.sync_copy(rows_v.at[b], acc_sh.at[dst_v.at[jj]], add=True)

    plsc.subcore_barrier()
    pltpu.sync_copy(acc_sh.at[pl.ds(base, ROWS_PER_TILE)],
                    out_hbm.at[c, pl.ds(base, ROWS_PER_TILE)])


def _mm_first_body(x_ref, c0_ref, c1_ref, w_ref, g_ref, dis_ref):
    deg = c0_ref[0] + c1_ref[0] + 1.0
    dis = lax.rsqrt(deg)
    dis_ref[...] = dis
    g_ref[...] = dis * jnp.dot(x_ref[...], w_ref[...],
                               preferred_element_type=jnp.float32)


_mm_first = pl.pallas_call(
    _mm_first_body,
    grid=(GRID,),
    in_specs=[
        pl.BlockSpec((BG, D), lambda i: (i, 0)),
        pl.BlockSpec((1, BG, 1), lambda i: (0, i, 0)),
        pl.BlockSpec((1, BG, 1), lambda i: (1, i, 0)),
        pl.BlockSpec((D, D), lambda i: (0, 0)),
    ],
    out_specs=[
        pl.BlockSpec((BG, D), lambda i: (i, 0)),
        pl.BlockSpec((BG, 1), lambda i: (i, 0)),
    ],
    out_shape=[
        jax.ShapeDtypeStruct((N_NODES, D), jnp.float32),
        jax.ShapeDtypeStruct((N_NODES, 1), jnp.float32),
    ],
)


def _mm_mid_body(z0_ref, z1_ref, dis_ref, b_ref, w_ref, g_ref, *, relu):
    dis = dis_ref[...]
    u = dis * (z0_ref[0] + z1_ref[0]) + b_ref[...]
    h = jnp.maximum(u, 0.0) if relu else u
    g_ref[...] = dis * jnp.dot(h, w_ref[...],
                               preferred_element_type=jnp.float32)


def _make_mm_mid(relu):
    return pl.pallas_call(
        functools.partial(_mm_mid_body, relu=relu),
        grid=(GRID,),
        in_specs=[
            pl.BlockSpec((1, BG, D), lambda i: (0, i, 0)),
            pl.BlockSpec((1, BG, D), lambda i: (1, i, 0)),
            pl.BlockSpec((BG, 1), lambda i: (i, 0)),
            pl.BlockSpec((1, D), lambda i: (0, 0)),
            pl.BlockSpec((D, D), lambda i: (0, 0)),
        ],
        out_specs=pl.BlockSpec((BG, D), lambda i: (i, 0)),
        out_shape=jax.ShapeDtypeStruct((N_NODES, D), jnp.float32),
    )


_mm_mid_relu = _make_mm_mid(True)
_mm_mid_lin = _make_mm_mid(False)


def _mm_last_body(z0_ref, z1_ref, dis_ref, b_ref, o_ref):
    o_ref[...] = dis_ref[...] * (z0_ref[0] + z1_ref[0]) + b_ref[...]


_mm_last = pl.pallas_call(
    _mm_last_body,
    grid=(GRID,),
    in_specs=[
        pl.BlockSpec((1, BG, D), lambda i: (0, i, 0)),
        pl.BlockSpec((1, BG, D), lambda i: (1, i, 0)),
        pl.BlockSpec((BG, 1), lambda i: (i, 0)),
        pl.BlockSpec((1, D), lambda i: (0, 0)),
    ],
    out_specs=pl.BlockSpec((BG, D), lambda i: (i, 0)),
    out_shape=jax.ShapeDtypeStruct((N_NODES, D), jnp.float32),
)


def kernel(x, edge_index, W1, b1, W2, b2):
    src = edge_index[0].astype(jnp.int32)
    dst = edge_index[1].astype(jnp.int32)
    npad = E_PAD - N_EDGES
    pidx = jnp.arange(npad, dtype=jnp.int32)
    src_full = jnp.concatenate([src, pidx % N_NODES])
    dst_full = jnp.concatenate([dst, N_NODES + pidx % JUNK])
    src3 = src_full.reshape(NW * 2, HC * K)
    dst3 = dst_full.reshape(NW * 2, HC, K)

    cnt = _deg_kernel(dst3)
    cnt3 = cnt[:, :, None]
    zeros = jnp.zeros((N_NODES, D), jnp.float32)
    b1r = b1.reshape(1, D)
    b2r = b2.reshape(1, D)

    g1, dis = _mm_first(x, cnt3, cnt3, W1)
    z1 = _agg_kernel(g1, zeros, src3, dst3)
    g2 = _mm_mid_relu(z1, z1, dis, b1r, W2)
    z2 = _agg_kernel(g2, zeros, src3, dst3)
    g3 = _mm_mid_lin(z2, z2, dis, b2r, W1)
    z3 = _agg_kernel(g3, zeros, src3, dst3)
    return _mm_last(z3, z3, dis, b1r)


# speedup vs baseline: 1.0431x; 1.0139x over previous
import functools

import jax
import jax.numpy as jnp
from jax import lax
from jax.experimental import pallas as pl
from jax.experimental.pallas import tpu as pltpu
from jax.experimental.pallas import tpu_sc as plsc

N_NODES = 10000
N_EDGES = 320000
D = 128

NC = 2
NS = 16
NW = NC * NS

K = 96
CHUNKS = 106
HC = CHUNKS // 2
NBUF = 3
E_PAD = NW * CHUNKS * K
N_PAD = 10112
JUNK = N_PAD - N_NODES
ROWS_PER_TILE = N_PAD // NS
DEG_BINS = 16384
DEG_PER_TILE = DEG_BINS // NS

LAST_ROWS = N_NODES - (NS - 1) * ROWS_PER_TILE

BG = 5000
GRID = N_NODES // BG

_sc_mesh = plsc.VectorSubcoreMesh(
    core_axis_name="c", subcore_axis_name="s", num_cores=NC, num_subcores=NS)


@functools.partial(
    pl.kernel,
    out_type=jax.ShapeDtypeStruct((NC, DEG_BINS), jnp.float32),
    mesh=_sc_mesh,
    scratch_types=[
        pltpu.VMEM((HC, K), jnp.int32),
        pltpu.VMEM((K,), jnp.float32),
        pltpu.VMEM((DEG_PER_TILE,), jnp.float32),
        pltpu.VMEM_SHARED((DEG_BINS,), jnp.float32),
    ],
)
def _deg_kernel(dst_hbm, out_hbm, dst_v, ones_v, z_v, hist_sh):
    c = lax.axis_index("c")
    s = lax.axis_index("s")
    wid = s * NC + c
    for i in range(K // 16):
        ones_v[pl.ds(i * 16, 16)] = jnp.ones((16,), jnp.float32)
    for i in range(DEG_PER_TILE // 16):
        z_v[pl.ds(i * 16, 16)] = jnp.zeros((16,), jnp.float32)
    pltpu.sync_copy(z_v, hist_sh.at[pl.ds(s * DEG_PER_TILE, DEG_PER_TILE)])
    plsc.subcore_barrier()

    for h in range(2):
        pltpu.sync_copy(dst_hbm.at[wid * 2 + h], dst_v)

        @pl.loop(0, HC)
        def _(j):
            pltpu.sync_copy(ones_v, hist_sh.at[dst_v.at[j]], add=True)

    plsc.subcore_barrier()
    pltpu.sync_copy(hist_sh.at[pl.ds(s * DEG_PER_TILE, DEG_PER_TILE)],
                    out_hbm.at[c, pl.ds(s * DEG_PER_TILE, DEG_PER_TILE)])


@functools.partial(
    pl.kernel,
    out_type=jax.ShapeDtypeStruct((NC, N_PAD, D), jnp.float32),
    mesh=_sc_mesh,
    scratch_types=[
        pltpu.VMEM((HC * K,), jnp.int32),
        pltpu.VMEM((HC, K), jnp.int32),
        pltpu.VMEM((NBUF, K, D), jnp.float32),
        pltpu.VMEM_SHARED((N_PAD, D), jnp.float32),
        pltpu.SemaphoreType.DMA,
        pltpu.SemaphoreType.DMA,
        pltpu.SemaphoreType.DMA,
    ],
)
def _agg_kernel(g_hbm, z_hbm, src_hbm, dst_hbm, out_hbm,
                src_v, dst_v, rows_v, acc_sh, gsem0, gsem1, gsem2):
    c = lax.axis_index("c")
    s = lax.axis_index("s")
    wid = s * NC + c
    base = s * ROWS_PER_TILE

    @pl.when((c == 0) & (s < NS - 1))
    def _():
        pltpu.sync_copy(g_hbm.at[pl.ds(base, ROWS_PER_TILE)],
                        acc_sh.at[pl.ds(base, ROWS_PER_TILE)])

    @pl.when((c == 0) & (s == NS - 1))
    def _():
        pltpu.sync_copy(g_hbm.at[pl.ds(base, LAST_ROWS)],
                        acc_sh.at[pl.ds(base, LAST_ROWS)])

    @pl.when((c != 0) & (s < NS - 1))
    def _():
        pltpu.sync_copy(z_hbm.at[pl.ds(base, ROWS_PER_TILE)],
                        acc_sh.at[pl.ds(base, ROWS_PER_TILE)])

    @pl.when((c != 0) & (s == NS - 1))
    def _():
        pltpu.sync_copy(z_hbm.at[pl.ds(base, LAST_ROWS)],
                        acc_sh.at[pl.ds(base, LAST_ROWS)])

    plsc.subcore_barrier()

    gsems = (gsem0, gsem1, gsem2)
    for h in range(2):
        pltpu.sync_copy(src_hbm.at[wid * 2 + h], src_v)
        pltpu.sync_copy(dst_hbm.at[wid * 2 + h], dst_v)
        for b in range(NBUF):
            pltpu.async_copy(
                g_hbm.at[src_v.at[pl.ds(b * K, K)]], rows_v.at[b], gsems[b])

        @pl.loop(0, HC - (HC % NBUF), step=NBUF)
        def _(j):
            for b in range(NBUF):
                jj = j + b
                pltpu.make_async_copy(
                    g_hbm.at[src_v.at[pl.ds(jj * K, K)]],
                    rows_v.at[b], gsems[b]).wait()
                pltpu.sync_copy(rows_v.at[b], acc_sh.at[dst_v.at[jj]], add=True)

                @pl.when(jj + NBUF < HC)
                def _():
                    pltpu.async_copy(
                        g_hbm.at[src_v.at[pl.ds((jj + NBUF) * K, K)]],
                        rows_v.at[b], gsems[b])

        for b in range(HC % NBUF):
            jj = HC - (HC % NBUF) + b
            pltpu.make_async_copy(
                g_hbm.at[src_v.at[pl.ds(jj * K, K)]],
                rows_v.at[b], gsems[b]).wait()
            pltpu.sync_copy(rows_v.at[b], acc_sh.at[dst_v.at[jj]], add=True)

    plsc.subcore_barrier()
    pltpu.sync_copy(acc_sh.at[pl.ds(base, ROWS_PER_TILE)],
                    out_hbm.at[c, pl.ds(base, ROWS_PER_TILE)])


def _mm_first_body(x_ref, c0_ref, c1_ref, w_ref, g_ref, dis_ref):
    deg = c0_ref[0] + c1_ref[0] + 1.0
    dis = lax.rsqrt(deg)
    dis_ref[...] = dis
    g_ref[...] = dis * jnp.dot(x_ref[...], w_ref[...],
                               preferred_element_type=jnp.float32)


_mm_first = pl.pallas_call(
    _mm_first_body,
    grid=(GRID,),
    in_specs=[
        pl.BlockSpec((BG, D), lambda i: (i, 0)),
        pl.BlockSpec((1, BG, 1), lambda i: (0, i, 0)),
        pl.BlockSpec((1, BG, 1), lambda i: (1, i, 0)),
        pl.BlockSpec((D, D), lambda i: (0, 0)),
    ],
    out_specs=[
        pl.BlockSpec((BG, D), lambda i: (i, 0)),
        pl.BlockSpec((BG, 1), lambda i: (i, 0)),
    ],
    out_shape=[
        jax.ShapeDtypeStruct((N_NODES, D), jnp.float32),
        jax.ShapeDtypeStruct((N_NODES, 1), jnp.float32),
    ],
)


def _mm_mid_body(z0_ref, z1_ref, dis_ref, b_ref, w_ref, g_ref, *, relu):
    dis = dis_ref[...]
    u = dis * (z0_ref[0] + z1_ref[0]) + b_ref[...]
    h = jnp.maximum(u, 0.0) if relu else u
    g_ref[...] = dis * jnp.dot(h, w_ref[...],
                               preferred_element_type=jnp.float32)


def _make_mm_mid(relu):
    return pl.pallas_call(
        functools.partial(_mm_mid_body, relu=relu),
        grid=(GRID,),
        in_specs=[
            pl.BlockSpec((1, BG, D), lambda i: (0, i, 0)),
            pl.BlockSpec((1, BG, D), lambda i: (1, i, 0)),
            pl.BlockSpec((BG, 1), lambda i: (i, 0)),
            pl.BlockSpec((1, D), lambda i: (0, 0)),
            pl.BlockSpec((D, D), lambda i: (0, 0)),
        ],
        out_specs=pl.BlockSpec((BG, D), lambda i: (i, 0)),
        out_shape=jax.ShapeDtypeStruct((N_NODES, D), jnp.float32),
    )


_mm_mid_relu = _make_mm_mid(True)
_mm_mid_lin = _make_mm_mid(False)


def _mm_last_body(z0_ref, z1_ref, dis_ref, b_ref, o_ref):
    o_ref[...] = dis_ref[...] * (z0_ref[0] + z1_ref[0]) + b_ref[...]


_mm_last = pl.pallas_call(
    _mm_last_body,
    grid=(GRID,),
    in_specs=[
        pl.BlockSpec((1, BG, D), lambda i: (0, i, 0)),
        pl.BlockSpec((1, BG, D), lambda i: (1, i, 0)),
        pl.BlockSpec((BG, 1), lambda i: (i, 0)),
        pl.BlockSpec((1, D), lambda i: (0, 0)),
    ],
    out_specs=pl.BlockSpec((BG, D), lambda i: (i, 0)),
    out_shape=jax.ShapeDtypeStruct((N_NODES, D), jnp.float32),
)


def kernel(x, edge_index, W1, b1, W2, b2):
    src = edge_index[0].astype(jnp.int32)
    dst = edge_index[1].astype(jnp.int32)
    npad = E_PAD - N_EDGES
    pidx = jnp.arange(npad, dtype=jnp.int32)
    src_full = jnp.concatenate([src, pidx % N_NODES])
    dst_full = jnp.concatenate([dst, N_NODES + pidx % JUNK])
    src3 = src_full.reshape(NW * 2, HC * K)
    dst3 = dst_full.reshape(NW * 2, HC, K)

    cnt = _deg_kernel(dst3)
    cnt3 = cnt[:, :, None]
    zeros = jnp.zeros((N_NODES, D), jnp.float32)
    b1r = b1.reshape(1, D)
    b2r = b2.reshape(1, D)

    g1, dis = _mm_first(x, cnt3, cnt3, W1)
    z1 = _agg_kernel(g1, zeros, src3, dst3)
    g2 = _mm_mid_relu(z1, z1, dis, b1r, W2)
    z2 = _agg_kernel(g2, zeros, src3, dst3)
    g3 = _mm_mid_lin(z2, z2, dis, b2r, W1)
    z3 = _agg_kernel(g3, zeros, src3, dst3)
    return _mm_last(z3, z3, dis, b1r)
